# Initial kernel scaffold; baseline (speedup 1.0000x reference)
import jax
import jax.numpy as jnp
from jax.experimental import pallas as pl


def kernel(x, w):
    raise NotImplementedError("write your pallas kernel here")


# speedup vs baseline: 1.0990x; 1.0990x over previous
import functools

import jax
import jax.numpy as jnp
from jax import lax
from jax.experimental import pallas as pl
from jax.experimental.pallas import tpu as pltpu
from jax.experimental.pallas import tpu_sc as plsc

EMB = 64
NUM = 1024
BM = 1024

_NC = 2
_NS = 16
_NW = _NC * _NS
_B = 16384
_BPW = _B // _NW


def _dist_argmin_body(x_ref, w_ref, idx_ref, wt_ref):
    pid = pl.program_id(0)
    xb = x_ref[...]
    w = w_ref[...]
    cross = jnp.dot(xb, w, preferred_element_type=jnp.float32)
    xsq = jnp.sum(xb * xb, axis=1, keepdims=True)
    wsq = jnp.sum(w * w, axis=0, keepdims=True)
    neg = -((xsq - 2.0 * cross) + wsq)
    m = jnp.max(neg, axis=1, keepdims=True)
    ids = lax.broadcasted_iota(jnp.int32, neg.shape, 1)
    idx = jnp.min(jnp.where(neg == m, ids, NUM), axis=1)
    idx_ref[0, 0, :] = idx

    @pl.when(pid == 0)
    def _():
        wt_ref[...] = w.T


def _gather_body(table_hbm, idx_hbm, out_hbm, idx_v, rows_v, sem):
    wid = lax.axis_index("s") * _NC + lax.axis_index("c")
    base = wid * _BPW
    pltpu.sync_copy(idx_hbm.at[pl.ds(base, _BPW)], idx_v)
    pltpu.async_copy(table_hbm.at[idx_v], rows_v, sem).wait()
    pltpu.sync_copy(rows_v, out_hbm.at[pl.ds(base, _BPW)])


def _sc_gather(wt, idx):
    mesh = plsc.VectorSubcoreMesh(core_axis_name="c", subcore_axis_name="s")
    return pl.kernel(
        _gather_body,
        mesh=mesh,
        compiler_params=pltpu.CompilerParams(use_tc_tiling_on_sc=False),
        out_type=jax.ShapeDtypeStruct((_B, EMB), jnp.float32),
        scratch_types=[
            pltpu.VMEM((_BPW,), jnp.int32),
            pltpu.VMEM((_BPW, EMB), jnp.float32),
            pltpu.SemaphoreType.DMA,
        ],
    )(wt, idx)


def kernel(x, w):
    xf = x.reshape(-1, EMB)
    grid = xf.shape[0] // BM
    idx3, wt = pl.pallas_call(
        _dist_argmin_body,
        grid=(grid,),
        in_specs=[
            pl.BlockSpec((BM, EMB), lambda i: (i, 0)),
            pl.BlockSpec((EMB, NUM), lambda i: (0, 0)),
        ],
        out_specs=[
            pl.BlockSpec((1, 1, BM), lambda i: (i, 0, 0)),
            pl.BlockSpec((NUM, EMB), lambda i: (0, 0)),
        ],
        out_shape=[
            jax.ShapeDtypeStruct((grid, 1, BM), jnp.int32),
            jax.ShapeDtypeStruct((NUM, EMB), jnp.float32),
        ],
    )(xf, w)
    idx = idx3.reshape(xf.shape[0])
    quantized = _sc_gather(wt, idx)
    return quantized.reshape(x.shape)


# speedup vs baseline: 1.2558x; 1.1426x over previous
import functools

import jax
import jax.numpy as jnp
from jax import lax
from jax.experimental import pallas as pl
from jax.experimental.pallas import tpu as pltpu
from jax.experimental.pallas import tpu_sc as plsc

EMB = 64
NUM = 1024
BN = 1024

_NC = 2
_NS = 16
_NW = _NC * _NS
_B = 16384
_BPW = _B // _NW
_CH = 128
_NCHUNK = _BPW // _CH


def _dist_argmin_body(wt_ref, xt_ref, wsq_ref, idx_ref):
    wt = wt_ref[...]
    xt = xt_ref[...]
    crossT = jnp.dot(wt, xt, preferred_element_type=jnp.float32)
    xsq = jnp.sum(xt * xt, axis=0, keepdims=True)
    dist = (xsq - 2.0 * crossT) + wsq_ref[...]
    m = jnp.min(dist, axis=0, keepdims=True)
    ids = lax.broadcasted_iota(jnp.int32, dist.shape, 0)
    idx_ref[0, 0, :] = jnp.min(jnp.where(dist == m, ids, NUM), axis=0)


def _gather_body(table_hbm, idx_hbm, out_hbm, idx_v, rows0, rows1, gsem, wsem):
    wid = lax.axis_index("s") * _NC + lax.axis_index("c")
    base = wid * _BPW
    pltpu.sync_copy(idx_hbm.at[pl.ds(base, _BPW)], idx_v)
    bufs = (rows0, rows1)
    copies = [None, None]
    for k in range(_NCHUNK):
        buf = bufs[k % 2]
        if copies[k % 2] is not None:
            copies[k % 2].wait()
        pltpu.async_copy(
            table_hbm.at[idx_v.at[pl.ds(k * _CH, _CH)]], buf, gsem
        ).wait()
        cp = pltpu.async_copy(buf, out_hbm.at[pl.ds(base + k * _CH, _CH)], wsem)
        copies[k % 2] = cp
    copies[0].wait()
    copies[1].wait()


def _sc_gather(wt, idx):
    mesh = plsc.VectorSubcoreMesh(core_axis_name="c", subcore_axis_name="s")
    return pl.kernel(
        _gather_body,
        mesh=mesh,
        compiler_params=pltpu.CompilerParams(use_tc_tiling_on_sc=False),
        out_type=jax.ShapeDtypeStruct((_B, EMB), jnp.float32),
        scratch_types=[
            pltpu.VMEM((_BPW,), jnp.int32),
            pltpu.VMEM((_CH, EMB), jnp.float32),
            pltpu.VMEM((_CH, EMB), jnp.float32),
            pltpu.SemaphoreType.DMA,
            pltpu.SemaphoreType.DMA,
        ],
    )(wt, idx)


def kernel(x, w):
    xf = x.reshape(-1, EMB)
    m = xf.shape[0]
    grid = m // BN
    xt = xf.T
    wt = w.T
    wsq = jnp.sum(w ** 2, axis=0, keepdims=True).T
    idx3 = pl.pallas_call(
        _dist_argmin_body,
        grid=(grid,),
        in_specs=[
            pl.BlockSpec((NUM, EMB), lambda i: (0, 0)),
            pl.BlockSpec((EMB, BN), lambda i: (0, i)),
            pl.BlockSpec((NUM, 1), lambda i: (0, 0)),
        ],
        out_specs=pl.BlockSpec((1, 1, BN), lambda i: (i, 0, 0)),
        out_shape=jax.ShapeDtypeStruct((grid, 1, BN), jnp.int32),
    )(wt, xt, wsq)
    idx = idx3.reshape(m)
    quantized = _sc_gather(wt, idx)
    return quantized.reshape(x.shape)
